# Initial kernel scaffold; baseline (speedup 1.0000x reference)
import jax
import jax.numpy as jnp
from jax.experimental import pallas as pl


def kernel(word_idx_list, W):
    raise NotImplementedError("write your pallas kernel here")


# speedup vs baseline: 5.2671x; 5.2671x over previous
import functools

import jax
import jax.numpy as jnp
from jax import lax
from jax.experimental import pallas as pl
from jax.experimental.pallas import tpu as pltpu
from jax.experimental.pallas import tpu_sc as plsc

VOCAB = 1000
EMBED = 64
BATCH = 16384
HIST = 50

_INFO = plsc.get_sparse_core_info()
_NC = _INFO.num_cores
_NS = _INFO.num_subcores
_NW = _NC * _NS

_B = BATCH * HIST
_B_PER_W = _B // _NW
_K = 4
_CHUNK = _K * 128
_NCHUNK = _B_PER_W // _CHUNK


def _make_kernel():
  mesh = plsc.VectorSubcoreMesh(core_axis_name="c", subcore_axis_name="s")

  @functools.partial(
      pl.kernel,
      mesh=mesh,
      compiler_params=pltpu.CompilerParams(use_tc_tiling_on_sc=False),
      out_type=jax.ShapeDtypeStruct((_B, EMBED), jnp.float32),
      scratch_types=[
          pltpu.VMEM((_K, 128), jnp.int32),
          pltpu.VMEM((_CHUNK, EMBED), jnp.float32),
          pltpu.SemaphoreType.DMA,
      ],
  )
  def gather_kernel(idx_hbm, table_hbm, out_hbm, idx_v, rows_v, gsem):
    wid = lax.axis_index("s") * _NC + lax.axis_index("c")

    def chunk_body(c, carry):
      pltpu.sync_copy(idx_hbm.at[wid, c], idx_v)
      copies = [
          pltpu.async_copy(
              table_hbm.at[idx_v.at[j]],
              rows_v.at[pl.ds(j * 128, 128)],
              gsem,
          )
          for j in range(_K)
      ]
      for cp in copies:
        cp.wait()
      pltpu.sync_copy(
          rows_v,
          out_hbm.at[pl.ds(wid * _B_PER_W + c * _CHUNK, _CHUNK)],
      )
      return carry

    lax.fori_loop(0, _NCHUNK, chunk_body, 0)

  return gather_kernel


_GATHER = _make_kernel()


def kernel(word_idx_list, W):
  idx = word_idx_list.astype(jnp.int32).reshape(_NW, _NCHUNK, _K, 128)
  out = _GATHER(idx, W)
  return out.reshape(BATCH, HIST, EMBED)


# speedup vs baseline: 5.2851x; 1.0034x over previous
import functools

import jax
import jax.numpy as jnp
from jax import lax
from jax.experimental import pallas as pl
from jax.experimental.pallas import tpu as pltpu
from jax.experimental.pallas import tpu_sc as plsc

VOCAB = 1000
EMBED = 64
BATCH = 16384
HIST = 50

_INFO = plsc.get_sparse_core_info()
_NC = _INFO.num_cores
_NS = _INFO.num_subcores
_NW = _NC * _NS

_B = BATCH * HIST
_B_PER_W = _B // _NW
_K = 4
_CHUNK = _K * 128
_NCHUNK = _B_PER_W // _CHUNK
_NPAIR = _NCHUNK // 2


def _make_kernel():
  mesh = plsc.VectorSubcoreMesh(core_axis_name="c", subcore_axis_name="s")

  @functools.partial(
      pl.kernel,
      mesh=mesh,
      compiler_params=pltpu.CompilerParams(use_tc_tiling_on_sc=False),
      out_type=jax.ShapeDtypeStruct((_B, EMBED), jnp.float32),
      scratch_types=[
          pltpu.VMEM((_NCHUNK * _K, 128), jnp.int32),
          pltpu.VMEM((2, _CHUNK, EMBED), jnp.float32),
          pltpu.SemaphoreType.DMA,
          pltpu.SemaphoreType.DMA,
          pltpu.SemaphoreType.DMA,
      ],
  )
  def gather_kernel(idx_hbm, table_hbm, out_hbm, idx_all, rows, gsem, s0, s1):
    wid = lax.axis_index("s") * _NC + lax.axis_index("c")
    base = wid * _B_PER_W
    ssems = (s0, s1)

    def run_gather(c, b):
      copies = [
          pltpu.async_copy(
              table_hbm.at[idx_all.at[c * _K + j]],
              rows.at[b].at[pl.ds(j * 128, 128)],
              gsem,
          )
          for j in range(_K)
      ]
      for cp in copies:
        cp.wait()

    def fire_store(c, b):
      pltpu.async_copy(
          rows.at[b], out_hbm.at[pl.ds(base + c * _CHUNK, _CHUNK)], ssems[b]
      )

    def wait_store(b):
      pltpu.make_async_copy(
          rows.at[b], out_hbm.at[pl.ds(0, _CHUNK)], ssems[b]
      ).wait()

    pltpu.sync_copy(idx_hbm.at[wid], idx_all)

    def pair_body(p, carry):
      for b in range(2):
        c = 2 * p + b

        @pl.when(c >= 2)
        def _():
          wait_store(b)

        run_gather(c, b)
        fire_store(c, b)
      return carry

    lax.fori_loop(0, _NPAIR, pair_body, 0)
    wait_store(0)
    wait_store(1)

  return gather_kernel


_GATHER = _make_kernel()


def kernel(word_idx_list, W):
  idx = word_idx_list.astype(jnp.int32).reshape(_NW, _NCHUNK * _K, 128)
  out = _GATHER(idx, W)
  return out.reshape(BATCH, HIST, EMBED)
